# baseline (speedup 1.0000x reference)
import functools

import jax
import jax.numpy as jnp
from jax import lax
from jax.experimental import pallas as pl
from jax.experimental.pallas import tpu as pltpu
from jax.experimental.pallas import tpu_sc as plsc

_CAPACITY = 1000000
_XDIM = 64
_BSZ = 16384

_NC = 2
_NS = 16
_NW = _NC * _NS
_BPW = _BSZ // _NW
_CHUNK = 128
_NCH = _BPW // _CHUNK


@functools.partial(
    pl.kernel,
    mesh=plsc.VectorSubcoreMesh(core_axis_name="c", subcore_axis_name="s"),
    compiler_params=pltpu.CompilerParams(use_tc_tiling_on_sc=False),
    out_type=(
        jax.ShapeDtypeStruct((_BSZ, _XDIM), jnp.float32),
        jax.ShapeDtypeStruct((_BSZ,), jnp.int32),
    ),
    scratch_types=[
        pltpu.VMEM((_NCH, _CHUNK), jnp.int32),
        pltpu.VMEM((_BPW, _XDIM), jnp.float32),
        pltpu.VMEM((_BPW,), jnp.int32),
        pltpu.SemaphoreType.DMA,
        pltpu.SemaphoreType.DMA,
    ],
)
def _fetch(mx_hbm, my_hbm, idx_hbm, outx_hbm, outy_hbm,
           idx_v, rows_v, lab_v, sem_x, sem_y):
    wid = lax.axis_index("s") * _NC + lax.axis_index("c")
    pltpu.sync_copy(idx_hbm.at[pl.ds(wid * _NCH, _NCH)], idx_v)
    copies = []
    for j in range(_NCH):
        copies.append(pltpu.async_copy(
            mx_hbm.at[idx_v.at[j]],
            rows_v.at[pl.ds(j * _CHUNK, _CHUNK)], sem_x))
        copies.append(pltpu.async_copy(
            my_hbm.at[idx_v.at[j]],
            lab_v.at[pl.ds(j * _CHUNK, _CHUNK)], sem_y))
    for c in copies:
        c.wait()
    base = wid * _BPW
    pltpu.sync_copy(rows_v, outx_hbm.at[pl.ds(base, _BPW)])
    pltpu.sync_copy(lab_v, outy_hbm.at[pl.ds(base, _BPW)])


def kernel(inputs, idx, mems_x, mems_y):
    del inputs
    idx2d = idx.reshape(_NW * _NCH, _CHUNK)
    return _fetch(mems_x, mems_y, idx2d)


# speedup vs baseline: 1.0004x; 1.0004x over previous
import functools

import jax
import jax.numpy as jnp
from jax import lax
from jax.experimental import pallas as pl
from jax.experimental.pallas import tpu as pltpu
from jax.experimental.pallas import tpu_sc as plsc

_CAPACITY = 1000000
_XDIM = 64
_BSZ = 16384

_NC = 2
_NS = 16
_NW = _NC * _NS
_BPW = _BSZ // _NW
_CHUNK = 128
_NCH = _BPW // _CHUNK


@functools.partial(
    pl.kernel,
    mesh=plsc.VectorSubcoreMesh(core_axis_name="c", subcore_axis_name="s"),
    compiler_params=pltpu.CompilerParams(use_tc_tiling_on_sc=False),
    out_type=(
        jax.ShapeDtypeStruct((_BSZ, _XDIM), jnp.float32),
        jax.ShapeDtypeStruct((_BSZ,), jnp.int32),
    ),
    scratch_types=[
        pltpu.VMEM((_NCH, _CHUNK), jnp.int32),
        pltpu.VMEM((_BPW, _XDIM), jnp.float32),
        pltpu.VMEM((_BPW,), jnp.int32),
        pltpu.SemaphoreType.DMA,
        pltpu.SemaphoreType.DMA,
    ],
)
def _fetch(mx_hbm, my_hbm, idx_hbm, outx_hbm, outy_hbm,
           idx_v, rows_v, lab_v, sem_x, sem_y):
    wid = lax.axis_index("s") * _NC + lax.axis_index("c")
    base_i = wid * _BPW
    for j in range(_NCH):
        pltpu.sync_copy(idx_hbm.at[pl.ds(base_i + j * _CHUNK, _CHUNK)],
                        idx_v.at[j])
    copies = []
    for j in range(_NCH):
        copies.append(pltpu.async_copy(
            mx_hbm.at[idx_v.at[j]],
            rows_v.at[pl.ds(j * _CHUNK, _CHUNK)], sem_x))
        copies.append(pltpu.async_copy(
            my_hbm.at[idx_v.at[j]],
            lab_v.at[pl.ds(j * _CHUNK, _CHUNK)], sem_y))
    for c in copies:
        c.wait()
    base = wid * _BPW
    pltpu.sync_copy(rows_v, outx_hbm.at[pl.ds(base, _BPW)])
    pltpu.sync_copy(lab_v, outy_hbm.at[pl.ds(base, _BPW)])


def kernel(inputs, idx, mems_x, mems_y):
    del inputs
    return _fetch(mems_x, mems_y, idx)
